# baseline (speedup 1.0000x reference)
import functools

import jax
import jax.numpy as jnp
from jax import lax
from jax.experimental import pallas as pl
from jax.experimental.pallas import tpu as pltpu
from jax.experimental.pallas import tpu_sc as plsc

N = 10000
DIN = 128
H = 64
B = 8
C = 8
NC = 2
NS = 16
K = 128
N_PAD = N + 8
RPW = 632
RPL = N - (NS - 1) * RPW


def _bf(v):
    return v.astype(jnp.bfloat16)


def _dot(a, b):
    return jnp.dot(_bf(a), _bf(b), preferred_element_type=jnp.float32)


NB = 4
LA = 2


def _make_agg(ch, d, k, ib):
    nb = ch // ib
    mesh = plsc.VectorSubcoreMesh(
        core_axis_name="c", subcore_axis_name="s", num_cores=NC,
        num_subcores=NS)

    @functools.partial(
        pl.kernel,
        out_type=jax.ShapeDtypeStruct((NC * N, d), jnp.float32),
        mesh=mesh,
        scratch_types=[
            pltpu.VMEM_SHARED((N_PAD, d), jnp.float32),
            pltpu.VMEM((2 * ib, k), jnp.int32),
            pltpu.VMEM((2 * ib, k), jnp.int32),
            pltpu.VMEM((NB, k, d), jnp.float32),
            pltpu.SemaphoreType.DMA,
            pltpu.SemaphoreType.DMA,
            pltpu.SemaphoreType.DMA((NB,)),
            pltpu.SemaphoreType.DMA((NB,)),
        ],
        compiler_params=pltpu.CompilerParams(use_tc_tiling_on_sc=False),
    )
    def agg(y_hbm, idx_hbm, out_hbm, acc, idx0, idx1, rows,
            semi0, semi1, semg, sems):
        c = lax.axis_index("c")
        s = lax.axis_index("s")
        wid = c * NS + s
        row0 = c * N + s * RPW

        @pl.when(s < NS - 1)
        def _():
            pltpu.sync_copy(y_hbm.at[pl.ds(row0, RPW)],
                            acc.at[pl.ds(s * RPW, RPW)])

        @pl.when(s == NS - 1)
        def _():
            pltpu.sync_copy(y_hbm.at[pl.ds(row0, RPL)],
                            acc.at[pl.ds(s * RPW, RPL)])

        pltpu.sync_copy(idx_hbm.at[wid, 0], idx0)
        if nb > 1:
            pltpu.async_copy(idx_hbm.at[wid, 1], idx1, semi1)
        for t in range(LA):
            pltpu.async_copy(y_hbm.at[idx0.at[2 * t]], rows.at[t],
                             semg.at[t])
        plsc.subcore_barrier()

        def wait_gather(b):
            pltpu.make_async_copy(y_hbm.at[idx0.at[0]], rows.at[b],
                                  semg.at[b]).wait()

        def wait_scatter(b):
            pltpu.make_async_copy(rows.at[b], acc.at[idx0.at[1]],
                                  sems.at[b]).wait()

        @pl.loop(0, ch)
        def _(j):
            jj = j + LA

            @pl.when(jj < ch)
            def _():
                jblk = jj // ib
                joff = jj % ib
                bjj = jj % NB

                @pl.when(joff == 0)
                def _():
                    @pl.when(jblk % 2 == 0)
                    def _():
                        pltpu.make_async_copy(idx_hbm.at[wid, jblk], idx0,
                                              semi0).wait()

                    @pl.when(jblk % 2 == 1)
                    def _():
                        pltpu.make_async_copy(idx_hbm.at[wid, jblk], idx1,
                                              semi1).wait()

                @pl.when(jj >= NB)
                def _():
                    wait_scatter(bjj)

                @pl.when(jblk % 2 == 0)
                def _():
                    pltpu.async_copy(y_hbm.at[idx0.at[2 * joff]],
                                     rows.at[bjj], semg.at[bjj])

                @pl.when(jblk % 2 == 1)
                def _():
                    pltpu.async_copy(y_hbm.at[idx1.at[2 * joff]],
                                     rows.at[bjj], semg.at[bjj])

                @pl.when((joff == LA + NB) & (jblk + 1 < nb))
                def _():
                    @pl.when(jblk % 2 == 0)
                    def _():
                        pltpu.async_copy(idx_hbm.at[wid, jblk + 1], idx1,
                                         semi1)

                    @pl.when(jblk % 2 == 1)
                    def _():
                        pltpu.async_copy(idx_hbm.at[wid, jblk + 1], idx0,
                                         semi0)

            b = j % NB
            off = j % ib
            wait_gather(b)

            @pl.when((j // ib) % 2 == 0)
            def _():
                pltpu.async_copy(rows.at[b], acc.at[idx0.at[2 * off + 1]],
                                 sems.at[b], add=True)

            @pl.when((j // ib) % 2 == 1)
            def _():
                pltpu.async_copy(rows.at[b], acc.at[idx1.at[2 * off + 1]],
                                 sems.at[b], add=True)

        for t in range(NB):
            wait_scatter(t)
        plsc.subcore_barrier()

        @pl.when(s < NS - 1)
        def _():
            pltpu.sync_copy(acc.at[pl.ds(s * RPW, RPW)],
                            out_hbm.at[pl.ds(row0, RPW)])

        @pl.when(s == NS - 1)
        def _():
            pltpu.sync_copy(acc.at[pl.ds(s * RPW, RPL)],
                            out_hbm.at[pl.ds(row0, RPL)])

    return agg


def _mlp1_body(a_ref, wa_ref, ba_ref, wb_ref, bb_ref, o_ref):
    u = jnp.maximum(_dot(a_ref[...], wa_ref[...]) + ba_ref[...], 0.0)
    o_ref[...] = jnp.maximum(_dot(u, wb_ref[...]) + bb_ref[...], 0.0)


def _mlp1(a, w1a, b1a, w1b, b1b):
    m = a.shape[0]
    r = 2000
    full = lambda i: (0, 0)
    return pl.pallas_call(
        _mlp1_body,
        grid=(m // r,),
        in_specs=[pl.BlockSpec((r, DIN), lambda i: (i, 0)),
                  pl.BlockSpec((DIN, H), full),
                  pl.BlockSpec((1, H), full),
                  pl.BlockSpec((H, H), full),
                  pl.BlockSpec((1, H), full)],
        out_specs=pl.BlockSpec((r, H), lambda i: (i, 0)),
        out_shape=jax.ShapeDtypeStruct((m, H), jnp.float32),
    )(a, w1a, b1a.reshape(1, H), w1b, b1b.reshape(1, H))


RT = 2000
NBG = N // RT
NGRID = 2 * NBG


def _tail_body(a_ref, w2a_ref, b2a_ref, w2b_ref, b2b_ref, batch_ref, wt_ref,
               wf1_ref, bf1_ref, wf2_ref, bf2_ref,
               wm1a_ref, wm1b_ref, wm1c_ref, bm1_ref,
               wm2_ref, bm2_ref, wm3_ref, bm3_ref,
               o_ref, sum_ref, cnt_ref):
    i = pl.program_id(0)
    g = i // NBG

    @pl.when(i == 0)
    def _():
        sum_ref[...] = jnp.zeros_like(sum_ref)
        cnt_ref[...] = jnp.zeros_like(cnt_ref)

    u = jnp.maximum(_dot(a_ref[...], w2a_ref[...]) + b2a_ref[...], 0.0)
    h2 = jnp.maximum(_dot(u, w2b_ref[...]) + b2b_ref[...], 0.0)
    batch_blk = batch_ref[0, 0, :]
    seg = lax.broadcasted_iota(jnp.int32, (2 * B, RT), 0)
    oh = (batch_blk[None, :] + g * B == seg).astype(jnp.float32)
    sum_ref[...] += jnp.dot(oh, h2, preferred_element_type=jnp.float32,
                            precision=jax.lax.Precision.HIGHEST)
    cnt_ref[...] += jnp.sum(oh, axis=1, keepdims=True)

    @pl.when(i == NGRID - 1)
    def _():
        gm = sum_ref[...] / jnp.maximum(cnt_ref[...], 1.0)
        g1 = gm[0:B]
        g2 = gm[B:2 * B]
        g2b = _bf(g2).astype(jnp.float32)
        s_mat = jnp.zeros((B, C), jnp.float32)
        col = lax.broadcasted_iota(jnp.int32, (1, C), 1)
        for cc in range(C):
            wc = wt_ref[cc * H:(cc + 1) * H, :]
            inter = _dot(g1, wc)
            sc = jnp.sum(_bf(inter).astype(jnp.float32) * g2b,
                         axis=1, keepdims=True)
            s_mat = s_mat + sc * (col == cc).astype(jnp.float32)
        s_mat = jnp.maximum(_dot(s_mat, wf1_ref[...]) + bf1_ref[...], 0.0)
        s_mat = jnp.maximum(_dot(s_mat, wf2_ref[...]) + bf2_ref[...], 0.0)
        h = (_dot(g1, wm1a_ref[...]) + _dot(g2, wm1b_ref[...])
             + _dot(s_mat, wm1c_ref[...]) + bm1_ref[...])
        h = jnp.maximum(h, 0.0)
        h = jnp.maximum(_dot(h, wm2_ref[...]) + bm2_ref[...], 0.0)
        out = _dot(h, wm3_ref[...]) + bm3_ref[...]
        o_ref[...] = out.reshape(1, B)


def _tail(a, w2a, b2a, w2b, b2b, batch3, wt2d, wf1, bf1, wf2, bf2,
          wm1, bm1, wm2, bm2, wm3, bm3):
    full = lambda i: (0, 0)
    return pl.pallas_call(
        _tail_body,
        grid=(NGRID,),
        in_specs=[
            pl.BlockSpec((RT, H), lambda i: (i, 0)),
            pl.BlockSpec((H, H), full),
            pl.BlockSpec((1, H), full),
            pl.BlockSpec((H, H), full),
            pl.BlockSpec((1, H), full),
            pl.BlockSpec((1, 1, RT), lambda i: (i, 0, 0)),
            pl.BlockSpec((C * H, H), full),
            pl.BlockSpec((C, C), full),
            pl.BlockSpec((1, C), full),
            pl.BlockSpec((C, C), full),
            pl.BlockSpec((1, C), full),
            pl.BlockSpec((H, H), full),
            pl.BlockSpec((H, H), full),
            pl.BlockSpec((C, H), full),
            pl.BlockSpec((1, H), full),
            pl.BlockSpec((H, H // 2), full),
            pl.BlockSpec((1, H // 2), full),
            pl.BlockSpec((H // 2, 1), full),
            pl.BlockSpec((1, 1), full),
        ],
        out_specs=pl.BlockSpec((1, B), full),
        out_shape=jax.ShapeDtypeStruct((1, B), jnp.float32),
        scratch_shapes=[pltpu.VMEM((2 * B, H), jnp.float32),
                        pltpu.VMEM((2 * B, 1), jnp.float32)],
    )(a, w2a, b2a.reshape(1, H), w2b, b2b.reshape(1, H), batch3, wt2d,
      wf1, bf1.reshape(1, C), wf2, bf2.reshape(1, C),
      wm1[:H], wm1[H:2 * H], wm1[2 * H:], bm1.reshape(1, H),
      wm2, bm2.reshape(1, H // 2), wm3, bm3.reshape(1, 1))


def kernel(x1, edge_index1, batch1, x2, edge_index2, batch2,
           W1a, b1a, W1b, b1b, W2a, b2a, W2b, b2b,
           Wt, Wf1, bf1, Wf2, bf2,
           Wm1, bm1, Wm2, bm2, Wm3, bm3):
    e = edge_index1.shape[1]
    ew = -(-e // NS)
    ib = 40
    ewp = -(-ew // (ib * K)) * (ib * K)
    pad = NS * ewp - e

    def prep(ei, g):
        src = jnp.concatenate(
            [ei[0] + g * N, jnp.full((pad,), g * N, jnp.int32)])
        dst = jnp.concatenate([ei[1], jnp.full((pad,), N, jnp.int32)])
        return src.reshape(NS, ewp), dst.reshape(NS, ewp)

    s1, d1 = prep(edge_index1, 0)
    s2, d2 = prep(edge_index2, 1)
    src_all = jnp.concatenate([s1, s2])
    dst_all = jnp.concatenate([d1, d2])

    def interleave(k):
        nb = ewp // k // ib
        s4 = src_all.reshape(NC * NS, nb, ib, 1, k)
        d4 = dst_all.reshape(NC * NS, nb, ib, 1, k)
        return jnp.concatenate([s4, d4], axis=3).reshape(
            NC * NS, nb, 2 * ib, k)

    x = jnp.concatenate([x1, x2])
    k1 = K // 2
    a1 = _make_agg(ewp // k1, DIN, k1, ib)(x, interleave(k1))
    h1 = _mlp1(a1, W1a, b1a, W1b, b1b)
    a2 = _make_agg(ewp // K, H, K, ib)(h1, interleave(K))
    batch3 = jnp.concatenate([batch1, batch2]).reshape(NGRID, 1, RT)
    out = _tail(a2, W2a, b2a, W2b, b2b, batch3, Wt.reshape(C * H, H),
                Wf1, bf1, Wf2, bf2, Wm1, bm1, Wm2, bm2, Wm3, bm3)
    return out.reshape(B)


# speedup vs baseline: 1.3026x; 1.3026x over previous
import functools

import jax
import jax.numpy as jnp
from jax import lax
from jax.experimental import pallas as pl
from jax.experimental.pallas import tpu as pltpu
from jax.experimental.pallas import tpu_sc as plsc

N = 10000
DIN = 128
H = 64
B = 8
C = 8
NC = 2
NS = 16
K = 128
N_PAD = N + 8
RPW = 632
RPL = N - (NS - 1) * RPW


def _bf(v):
    return v.astype(jnp.bfloat16)


def _dot(a, b):
    return jnp.dot(_bf(a), _bf(b), preferred_element_type=jnp.float32)


def _make_agg(ch, d, k):
    mesh = plsc.VectorSubcoreMesh(
        core_axis_name="c", subcore_axis_name="s", num_cores=NC,
        num_subcores=NS)

    @functools.partial(
        pl.kernel,
        out_type=jax.ShapeDtypeStruct((NC * N, d), jnp.float32),
        mesh=mesh,
        scratch_types=[
            pltpu.VMEM_SHARED((N_PAD, d), jnp.float32),
            pltpu.VMEM((ch, k), jnp.int32),
            pltpu.VMEM((ch, k), jnp.int32),
            pltpu.VMEM((k, d), jnp.float32),
            pltpu.VMEM((k, d), jnp.float32),
            pltpu.SemaphoreType.DMA,
            pltpu.SemaphoreType.DMA,
        ],
        compiler_params=pltpu.CompilerParams(use_tc_tiling_on_sc=False),
    )
    def agg(y_hbm, src_hbm, dst_hbm, out_hbm, acc, src_v, dst_v,
            rows0, rows1, sem0, sem1):
        c = lax.axis_index("c")
        s = lax.axis_index("s")
        wid = c * NS + s
        row0 = c * N + s * RPW

        @pl.when(s < NS - 1)
        def _():
            pltpu.sync_copy(y_hbm.at[pl.ds(row0, RPW)],
                            acc.at[pl.ds(s * RPW, RPW)])

        @pl.when(s == NS - 1)
        def _():
            pltpu.sync_copy(y_hbm.at[pl.ds(row0, RPL)],
                            acc.at[pl.ds(s * RPW, RPL)])
        pltpu.sync_copy(src_hbm.at[wid], src_v)
        pltpu.sync_copy(dst_hbm.at[wid], dst_v)
        plsc.subcore_barrier()
        pltpu.async_copy(y_hbm.at[src_v.at[0]], rows0, sem0)
        pltpu.async_copy(y_hbm.at[src_v.at[1]], rows1, sem1)

        @pl.loop(0, ch, step=2)
        def _(j):
            pltpu.make_async_copy(y_hbm.at[src_v.at[j]], rows0, sem0).wait()
            pltpu.sync_copy(rows0, acc.at[dst_v.at[j]], add=True)

            @pl.when(j + 2 < ch)
            def _():
                pltpu.async_copy(y_hbm.at[src_v.at[j + 2]], rows0, sem0)

            pltpu.make_async_copy(
                y_hbm.at[src_v.at[j + 1]], rows1, sem1).wait()
            pltpu.sync_copy(rows1, acc.at[dst_v.at[j + 1]], add=True)

            @pl.when(j + 3 < ch)
            def _():
                pltpu.async_copy(y_hbm.at[src_v.at[j + 3]], rows1, sem1)

        plsc.subcore_barrier()

        @pl.when(s < NS - 1)
        def _():
            pltpu.sync_copy(acc.at[pl.ds(s * RPW, RPW)],
                            out_hbm.at[pl.ds(row0, RPW)])

        @pl.when(s == NS - 1)
        def _():
            pltpu.sync_copy(acc.at[pl.ds(s * RPW, RPL)],
                            out_hbm.at[pl.ds(row0, RPL)])

    return agg


def _mlp1_body(a_ref, wa_ref, ba_ref, wb_ref, bb_ref, o_ref):
    u = jnp.maximum(_dot(a_ref[...], wa_ref[...]) + ba_ref[...], 0.0)
    o_ref[...] = jnp.maximum(_dot(u, wb_ref[...]) + bb_ref[...], 0.0)


def _mlp1(a, w1a, b1a, w1b, b1b):
    m = a.shape[0]
    r = 2000
    full = lambda i: (0, 0)
    return pl.pallas_call(
        _mlp1_body,
        grid=(m // r,),
        in_specs=[pl.BlockSpec((r, DIN), lambda i: (i, 0)),
                  pl.BlockSpec((DIN, H), full),
                  pl.BlockSpec((1, H), full),
                  pl.BlockSpec((H, H), full),
                  pl.BlockSpec((1, H), full)],
        out_specs=pl.BlockSpec((r, H), lambda i: (i, 0)),
        out_shape=jax.ShapeDtypeStruct((m, H), jnp.float32),
    )(a, w1a, b1a.reshape(1, H), w1b, b1b.reshape(1, H))


RT = 2000
NBG = N // RT
NGRID = 2 * NBG


def _tail_body(a_ref, w2a_ref, b2a_ref, w2b_ref, b2b_ref, batch_ref, wt_ref,
               wf1_ref, bf1_ref, wf2_ref, bf2_ref,
               wm1a_ref, wm1b_ref, wm1c_ref, bm1_ref,
               wm2_ref, bm2_ref, wm3_ref, bm3_ref,
               o_ref, sum_ref, cnt_ref):
    i = pl.program_id(0)
    g = i // NBG

    @pl.when(i == 0)
    def _():
        sum_ref[...] = jnp.zeros_like(sum_ref)
        cnt_ref[...] = jnp.zeros_like(cnt_ref)

    u = jnp.maximum(_dot(a_ref[...], w2a_ref[...]) + b2a_ref[...], 0.0)
    h2 = jnp.maximum(_dot(u, w2b_ref[...]) + b2b_ref[...], 0.0)
    batch_blk = batch_ref[0, 0, :]
    seg = lax.broadcasted_iota(jnp.int32, (2 * B, RT), 0)
    oh = (batch_blk[None, :] + g * B == seg).astype(jnp.float32)
    sum_ref[...] += jnp.dot(oh, h2, preferred_element_type=jnp.float32,
                            precision=jax.lax.Precision.HIGHEST)
    cnt_ref[...] += jnp.sum(oh, axis=1, keepdims=True)

    @pl.when(i == NGRID - 1)
    def _():
        gm = sum_ref[...] / jnp.maximum(cnt_ref[...], 1.0)
        g1 = gm[0:B]
        g2 = gm[B:2 * B]
        g2b = _bf(g2).astype(jnp.float32)
        s_mat = jnp.zeros((B, C), jnp.float32)
        col = lax.broadcasted_iota(jnp.int32, (1, C), 1)
        for cc in range(C):
            wc = wt_ref[cc * H:(cc + 1) * H, :]
            inter = _dot(g1, wc)
            sc = jnp.sum(_bf(inter).astype(jnp.float32) * g2b,
                         axis=1, keepdims=True)
            s_mat = s_mat + sc * (col == cc).astype(jnp.float32)
        s_mat = jnp.maximum(_dot(s_mat, wf1_ref[...]) + bf1_ref[...], 0.0)
        s_mat = jnp.maximum(_dot(s_mat, wf2_ref[...]) + bf2_ref[...], 0.0)
        h = (_dot(g1, wm1a_ref[...]) + _dot(g2, wm1b_ref[...])
             + _dot(s_mat, wm1c_ref[...]) + bm1_ref[...])
        h = jnp.maximum(h, 0.0)
        h = jnp.maximum(_dot(h, wm2_ref[...]) + bm2_ref[...], 0.0)
        out = _dot(h, wm3_ref[...]) + bm3_ref[...]
        o_ref[...] = out.reshape(1, B)


def _tail(a, w2a, b2a, w2b, b2b, batch3, wt2d, wf1, bf1, wf2, bf2,
          wm1, bm1, wm2, bm2, wm3, bm3):
    full = lambda i: (0, 0)
    return pl.pallas_call(
        _tail_body,
        grid=(NGRID,),
        in_specs=[
            pl.BlockSpec((RT, H), lambda i: (i, 0)),
            pl.BlockSpec((H, H), full),
            pl.BlockSpec((1, H), full),
            pl.BlockSpec((H, H), full),
            pl.BlockSpec((1, H), full),
            pl.BlockSpec((1, 1, RT), lambda i: (i, 0, 0)),
            pl.BlockSpec((C * H, H), full),
            pl.BlockSpec((C, C), full),
            pl.BlockSpec((1, C), full),
            pl.BlockSpec((C, C), full),
            pl.BlockSpec((1, C), full),
            pl.BlockSpec((H, H), full),
            pl.BlockSpec((H, H), full),
            pl.BlockSpec((C, H), full),
            pl.BlockSpec((1, H), full),
            pl.BlockSpec((H, H // 2), full),
            pl.BlockSpec((1, H // 2), full),
            pl.BlockSpec((H // 2, 1), full),
            pl.BlockSpec((1, 1), full),
        ],
        out_specs=pl.BlockSpec((1, B), full),
        out_shape=jax.ShapeDtypeStruct((1, B), jnp.float32),
        scratch_shapes=[pltpu.VMEM((2 * B, H), jnp.float32),
                        pltpu.VMEM((2 * B, 1), jnp.float32)],
    )(a, w2a, b2a.reshape(1, H), w2b, b2b.reshape(1, H), batch3, wt2d,
      wf1, bf1.reshape(1, C), wf2, bf2.reshape(1, C),
      wm1[:H], wm1[H:2 * H], wm1[2 * H:], bm1.reshape(1, H),
      wm2, bm2.reshape(1, H // 2), wm3, bm3.reshape(1, 1))


def kernel(x1, edge_index1, batch1, x2, edge_index2, batch2,
           W1a, b1a, W1b, b1b, W2a, b2a, W2b, b2b,
           Wt, Wf1, bf1, Wf2, bf2,
           Wm1, bm1, Wm2, bm2, Wm3, bm3):
    e = edge_index1.shape[1]
    ew = -(-e // NS)
    k1 = 32
    lcmk = 2 * K
    ewp = -(-ew // lcmk) * lcmk
    pad = NS * ewp - e

    def prep(ei, g):
        src = jnp.concatenate(
            [ei[0] + g * N, jnp.full((pad,), g * N, jnp.int32)])
        dst = jnp.concatenate([ei[1], jnp.full((pad,), N, jnp.int32)])
        return src.reshape(NS, ewp), dst.reshape(NS, ewp)

    s1, d1 = prep(edge_index1, 0)
    s2, d2 = prep(edge_index2, 1)
    src_all = jnp.concatenate([s1, s2])
    dst_all = jnp.concatenate([d1, d2])
    rs = lambda a, k: a.reshape(NC * NS, ewp // k, k)

    x = jnp.concatenate([x1, x2])
    a1 = _make_agg(ewp // k1, DIN, k1)(
        x, rs(src_all, k1), rs(dst_all, k1))
    h1 = _mlp1(a1, W1a, b1a, W1b, b1b)
    a2 = _make_agg(ewp // K, H, K)(
        h1, rs(src_all, K), rs(dst_all, K))
    batch3 = jnp.concatenate([batch1, batch2]).reshape(NGRID, 1, RT)
    out = _tail(a2, W2a, b2a, W2b, b2b, batch3, Wt.reshape(C * H, H),
                Wf1, bf1, Wf2, bf2, Wm1, bm1, Wm2, bm2, Wm3, bm3)
    return out.reshape(B)


# speedup vs baseline: 1.7770x; 1.3642x over previous
import functools

import jax
import jax.numpy as jnp
from jax import lax
from jax.experimental import pallas as pl
from jax.experimental.pallas import tpu as pltpu
from jax.experimental.pallas import tpu_sc as plsc

N = 10000
DIN = 128
H = 64
B = 8
C = 8
NC = 2
NS = 16
K = 128
N_PAD = N + 8
RPW = 632
RPL = N - (NS - 1) * RPW


def _bf(v):
    return v.astype(jnp.bfloat16)


def _dot(a, b):
    return jnp.dot(_bf(a), _bf(b), preferred_element_type=jnp.float32)


K1 = 32


def _make_agg1(c1):
    mesh = plsc.VectorSubcoreMesh(
        core_axis_name="c", subcore_axis_name="s", num_cores=NC,
        num_subcores=NS)

    @functools.partial(
        pl.kernel,
        out_type=jax.ShapeDtypeStruct((NC * N, DIN), jnp.float32),
        mesh=mesh,
        scratch_types=[
            pltpu.VMEM_SHARED((N_PAD, DIN), jnp.float32),
            pltpu.VMEM((c1, K1), jnp.int32),
            pltpu.VMEM((c1, K1), jnp.int32),
            pltpu.VMEM((K1, DIN), jnp.float32),
            pltpu.VMEM((K1, DIN), jnp.float32),
            pltpu.SemaphoreType.DMA,
            pltpu.SemaphoreType.DMA,
        ],
        compiler_params=pltpu.CompilerParams(use_tc_tiling_on_sc=False),
    )
    def agg(x1_hbm, x2_hbm, e1_hbm, e2_hbm, out_hbm, acc, src_v, dst_v,
            rows0, rows1, sem0, sem1):
        c = lax.axis_index("c")
        s = lax.axis_index("s")
        row0 = c * N + s * RPW

        def run(xref, eref):
            @pl.when(s < NS - 1)
            def _():
                pltpu.sync_copy(xref.at[pl.ds(s * RPW, RPW)],
                                acc.at[pl.ds(s * RPW, RPW)])

            @pl.when(s == NS - 1)
            def _():
                pltpu.sync_copy(xref.at[pl.ds(s * RPW, RPL)],
                                acc.at[pl.ds(s * RPW, RPL)])
            pltpu.sync_copy(eref.at[0, s], src_v)
            pltpu.sync_copy(eref.at[1, s], dst_v)
            plsc.subcore_barrier()
            pltpu.async_copy(xref.at[src_v.at[0]], rows0, sem0)
            pltpu.async_copy(xref.at[src_v.at[1]], rows1, sem1)

            @pl.loop(0, c1 - (c1 % 2), step=2)
            def _(j):
                pltpu.make_async_copy(
                    xref.at[src_v.at[j]], rows0, sem0).wait()
                pltpu.sync_copy(rows0, acc.at[dst_v.at[j]], add=True)

                @pl.when(j + 2 < c1)
                def _():
                    pltpu.async_copy(xref.at[src_v.at[j + 2]], rows0, sem0)

                pltpu.make_async_copy(
                    xref.at[src_v.at[j + 1]], rows1, sem1).wait()
                pltpu.sync_copy(rows1, acc.at[dst_v.at[j + 1]], add=True)

                @pl.when(j + 3 < c1)
                def _():
                    pltpu.async_copy(xref.at[src_v.at[j + 3]], rows1, sem1)

            if c1 % 2:
                pltpu.make_async_copy(
                    xref.at[src_v.at[c1 - 1]], rows0, sem0).wait()
                pltpu.sync_copy(rows0, acc.at[dst_v.at[c1 - 1]], add=True)
            plsc.subcore_barrier()

            @pl.when(s < NS - 1)
            def _():
                pltpu.sync_copy(acc.at[pl.ds(s * RPW, RPW)],
                                out_hbm.at[pl.ds(row0, RPW)])

            @pl.when(s == NS - 1)
            def _():
                pltpu.sync_copy(acc.at[pl.ds(s * RPW, RPL)],
                                out_hbm.at[pl.ds(row0, RPL)])

        @pl.when(c == 0)
        def _():
            run(x1_hbm, e1_hbm)

        @pl.when(c == 1)
        def _():
            run(x2_hbm, e2_hbm)

    return agg


def _make_agg(ch, d, k):
    mesh = plsc.VectorSubcoreMesh(
        core_axis_name="c", subcore_axis_name="s", num_cores=NC,
        num_subcores=NS)

    @functools.partial(
        pl.kernel,
        out_type=jax.ShapeDtypeStruct((NC * N, d), jnp.float32),
        mesh=mesh,
        scratch_types=[
            pltpu.VMEM_SHARED((N_PAD, d), jnp.float32),
            pltpu.VMEM((ch, k), jnp.int32),
            pltpu.VMEM((ch, k), jnp.int32),
            pltpu.VMEM((k, d), jnp.float32),
            pltpu.VMEM((k, d), jnp.float32),
            pltpu.SemaphoreType.DMA,
            pltpu.SemaphoreType.DMA,
        ],
        compiler_params=pltpu.CompilerParams(use_tc_tiling_on_sc=False),
    )
    def agg(y_hbm, src_hbm, dst_hbm, out_hbm, acc, src_v, dst_v,
            rows0, rows1, sem0, sem1):
        c = lax.axis_index("c")
        s = lax.axis_index("s")
        wid = c * NS + s
        row0 = c * N + s * RPW

        @pl.when(s < NS - 1)
        def _():
            pltpu.sync_copy(y_hbm.at[pl.ds(row0, RPW)],
                            acc.at[pl.ds(s * RPW, RPW)])

        @pl.when(s == NS - 1)
        def _():
            pltpu.sync_copy(y_hbm.at[pl.ds(row0, RPL)],
                            acc.at[pl.ds(s * RPW, RPL)])
        pltpu.sync_copy(src_hbm.at[wid], src_v)
        pltpu.sync_copy(dst_hbm.at[wid], dst_v)
        plsc.subcore_barrier()
        pltpu.async_copy(y_hbm.at[src_v.at[0]], rows0, sem0)
        pltpu.async_copy(y_hbm.at[src_v.at[1]], rows1, sem1)

        @pl.loop(0, ch, step=2)
        def _(j):
            pltpu.make_async_copy(y_hbm.at[src_v.at[j]], rows0, sem0).wait()
            pltpu.sync_copy(rows0, acc.at[dst_v.at[j]], add=True)

            @pl.when(j + 2 < ch)
            def _():
                pltpu.async_copy(y_hbm.at[src_v.at[j + 2]], rows0, sem0)

            pltpu.make_async_copy(
                y_hbm.at[src_v.at[j + 1]], rows1, sem1).wait()
            pltpu.sync_copy(rows1, acc.at[dst_v.at[j + 1]], add=True)

            @pl.when(j + 3 < ch)
            def _():
                pltpu.async_copy(y_hbm.at[src_v.at[j + 3]], rows1, sem1)

        plsc.subcore_barrier()

        @pl.when(s < NS - 1)
        def _():
            pltpu.sync_copy(acc.at[pl.ds(s * RPW, RPW)],
                            out_hbm.at[pl.ds(row0, RPW)])

        @pl.when(s == NS - 1)
        def _():
            pltpu.sync_copy(acc.at[pl.ds(s * RPW, RPL)],
                            out_hbm.at[pl.ds(row0, RPL)])

    return agg


def _mlp1_body(a_ref, wa_ref, ba_ref, wb_ref, bb_ref, o_ref):
    u = jnp.maximum(_dot(a_ref[...], wa_ref[...]) + ba_ref[...], 0.0)
    o_ref[...] = jnp.maximum(_dot(u, wb_ref[...]) + bb_ref[...], 0.0)


def _mlp1(a, w1a, b1a, w1b, b1b):
    m = a.shape[0]
    r = 2000
    full = lambda i: (0, 0)
    return pl.pallas_call(
        _mlp1_body,
        grid=(m // r,),
        in_specs=[pl.BlockSpec((r, DIN), lambda i: (i, 0)),
                  pl.BlockSpec((DIN, H), full),
                  pl.BlockSpec((1, H), full),
                  pl.BlockSpec((H, H), full),
                  pl.BlockSpec((1, H), full)],
        out_specs=pl.BlockSpec((r, H), lambda i: (i, 0)),
        out_shape=jax.ShapeDtypeStruct((m, H), jnp.float32),
    )(a, w1a, b1a.reshape(1, H), w1b, b1b.reshape(1, H))


RT = 2000
NBG = N // RT
NGRID = 2 * NBG


def _tail_body(a_ref, w2a_ref, b2a_ref, w2b_ref, b2b_ref, batch_ref, wt_ref,
               wf1_ref, bf1_ref, wf2_ref, bf2_ref,
               wm1a_ref, wm1b_ref, wm1c_ref, bm1_ref,
               wm2_ref, bm2_ref, wm3_ref, bm3_ref,
               o_ref, sum_ref, cnt_ref):
    i = pl.program_id(0)
    g = i // NBG

    @pl.when(i == 0)
    def _():
        sum_ref[...] = jnp.zeros_like(sum_ref)
        cnt_ref[...] = jnp.zeros_like(cnt_ref)

    u = jnp.maximum(_dot(a_ref[...], w2a_ref[...]) + b2a_ref[...], 0.0)
    h2 = jnp.maximum(_dot(u, w2b_ref[...]) + b2b_ref[...], 0.0)
    batch_blk = batch_ref[0, 0, :]
    seg = lax.broadcasted_iota(jnp.int32, (2 * B, RT), 0)
    oh = (batch_blk[None, :] + g * B == seg).astype(jnp.float32)
    sum_ref[...] += jnp.dot(oh, h2, preferred_element_type=jnp.float32,
                            precision=jax.lax.Precision.HIGHEST)
    cnt_ref[...] += jnp.sum(oh, axis=1, keepdims=True)

    @pl.when(i == NGRID - 1)
    def _():
        gm = sum_ref[...] / jnp.maximum(cnt_ref[...], 1.0)
        g1 = gm[0:B]
        g2 = gm[B:2 * B]
        g2b = _bf(g2).astype(jnp.float32)
        s_mat = jnp.zeros((B, C), jnp.float32)
        col = lax.broadcasted_iota(jnp.int32, (1, C), 1)
        for cc in range(C):
            wc = wt_ref[cc * H:(cc + 1) * H, :]
            inter = _dot(g1, wc)
            sc = jnp.sum(_bf(inter).astype(jnp.float32) * g2b,
                         axis=1, keepdims=True)
            s_mat = s_mat + sc * (col == cc).astype(jnp.float32)
        s_mat = jnp.maximum(_dot(s_mat, wf1_ref[...]) + bf1_ref[...], 0.0)
        s_mat = jnp.maximum(_dot(s_mat, wf2_ref[...]) + bf2_ref[...], 0.0)
        h = (_dot(g1, wm1a_ref[...]) + _dot(g2, wm1b_ref[...])
             + _dot(s_mat, wm1c_ref[...]) + bm1_ref[...])
        h = jnp.maximum(h, 0.0)
        h = jnp.maximum(_dot(h, wm2_ref[...]) + bm2_ref[...], 0.0)
        out = _dot(h, wm3_ref[...]) + bm3_ref[...]
        o_ref[...] = out.reshape(1, B)


def _tail(a, w2a, b2a, w2b, b2b, batch3, wt2d, wf1, bf1, wf2, bf2,
          wm1, bm1, wm2, bm2, wm3, bm3):
    full = lambda i: (0, 0)
    return pl.pallas_call(
        _tail_body,
        grid=(NGRID,),
        in_specs=[
            pl.BlockSpec((RT, H), lambda i: (i, 0)),
            pl.BlockSpec((H, H), full),
            pl.BlockSpec((1, H), full),
            pl.BlockSpec((H, H), full),
            pl.BlockSpec((1, H), full),
            pl.BlockSpec((1, 1, RT), lambda i: (i, 0, 0)),
            pl.BlockSpec((C * H, H), full),
            pl.BlockSpec((C, C), full),
            pl.BlockSpec((1, C), full),
            pl.BlockSpec((C, C), full),
            pl.BlockSpec((1, C), full),
            pl.BlockSpec((H, H), full),
            pl.BlockSpec((H, H), full),
            pl.BlockSpec((C, H), full),
            pl.BlockSpec((1, H), full),
            pl.BlockSpec((H, H // 2), full),
            pl.BlockSpec((1, H // 2), full),
            pl.BlockSpec((H // 2, 1), full),
            pl.BlockSpec((1, 1), full),
        ],
        out_specs=pl.BlockSpec((1, B), full),
        out_shape=jax.ShapeDtypeStruct((1, B), jnp.float32),
        scratch_shapes=[pltpu.VMEM((2 * B, H), jnp.float32),
                        pltpu.VMEM((2 * B, 1), jnp.float32)],
    )(a, w2a, b2a.reshape(1, H), w2b, b2b.reshape(1, H), batch3, wt2d,
      wf1, bf1.reshape(1, C), wf2, bf2.reshape(1, C),
      wm1[:H], wm1[H:2 * H], wm1[2 * H:], bm1.reshape(1, H),
      wm2, bm2.reshape(1, H // 2), wm3, bm3.reshape(1, 1))


def kernel(x1, edge_index1, batch1, x2, edge_index2, batch2,
           W1a, b1a, W1b, b1b, W2a, b2a, W2b, b2b,
           Wt, Wf1, bf1, Wf2, bf2,
           Wm1, bm1, Wm2, bm2, Wm3, bm3):
    e = edge_index1.shape[1]
    ew = e // NS
    c1 = ew // K1
    ewp = -(-ew // (2 * K)) * (2 * K)
    pad = NS * ewp - e

    def prep(ei, g):
        src = jnp.concatenate(
            [ei[0] + g * N, jnp.full((pad,), g * N, jnp.int32)])
        dst = jnp.concatenate([ei[1], jnp.full((pad,), N, jnp.int32)])
        return src.reshape(NS, ewp), dst.reshape(NS, ewp)

    s1, d1 = prep(edge_index1, 0)
    s2, d2 = prep(edge_index2, 1)
    src_all = jnp.concatenate([s1, s2])
    dst_all = jnp.concatenate([d1, d2])
    rs = lambda a: a.reshape(NC * NS, ewp // K, K)

    a1 = _make_agg1(c1)(
        x1, x2, edge_index1.reshape(2, NS, c1, K1),
        edge_index2.reshape(2, NS, c1, K1))
    h1 = _mlp1(a1, W1a, b1a, W1b, b1b)
    a2 = _make_agg(ewp // K, H, K)(
        h1, rs(src_all), rs(dst_all))
    batch3 = jnp.concatenate([batch1, batch2]).reshape(NGRID, 1, RT)
    out = _tail(a2, W2a, b2a, W2b, b2b, batch3, Wt.reshape(C * H, H),
                Wf1, bf1, Wf2, bf2, Wm1, bm1, Wm2, bm2, Wm3, bm3)
    return out.reshape(B)
